# Initial kernel scaffold; baseline (speedup 1.0000x reference)
import jax
import jax.numpy as jnp
from jax.experimental import pallas as pl


def kernel(h, edge_index, W_self1, W_neigh1, b1, W_self2, W_neigh2, b2):
    raise NotImplementedError("write your pallas kernel here")


# speedup vs baseline: 2.5301x; 2.5301x over previous
import jax
import jax.numpy as jnp
from jax import lax
from jax.experimental import pallas as pl
from jax.experimental.pallas import tpu as pltpu
from jax.experimental.pallas import tpu_sc as plsc

N_NODES = 10000
N_EDGES = 320000
D = 128

NC = 2
NS = 16
NW = NC * NS
E_PER_W = N_EDGES // NW
E_W_PAD = 10240
CHUNK = 16
NCHUNK = E_W_PAD // CHUNK
N_PAD = 10240
SEG = N_PAD // NS
NBLK = SEG // CHUNK


def _sc_agg_body(with_deg, x_hbm, packed_hbm, agg_out, deg_out,
                 packed_v, buf0, deg_v, dbuf, shared_agg, shared_deg, sem0):
  cid = lax.axis_index("c")
  sid = lax.axis_index("s")
  wid = cid * NS + sid
  row0 = sid * SEG

  zero16 = jnp.zeros((CHUNK,), jnp.float32)

  def zero_rows(r, carry):
    for c in range(D // CHUNK):
      buf0[r, pl.ds(c * CHUNK, CHUNK)] = zero16
    return carry

  lax.fori_loop(0, CHUNK, zero_rows, 0)

  def init_blk(k, carry):
    pltpu.sync_copy(buf0, shared_agg.at[pl.ds(row0 + k * CHUNK, CHUNK)])
    return carry

  lax.fori_loop(0, NBLK, init_blk, 0)

  if with_deg:
    def zero_deg(i, carry):
      deg_v[pl.ds(i * CHUNK, CHUNK)] = zero16
      return carry

    lax.fori_loop(0, N_PAD // CHUNK, zero_deg, 0)

  pltpu.sync_copy(packed_hbm.at[wid], packed_v)

  plsc.subcore_barrier()

  one16 = jnp.full((CHUNK,), 1.0, jnp.float32)

  def step(j, carry):
    v = packed_v[pl.ds(j * CHUNK, CHUNK)]
    s = v & 0xFFFF
    d = lax.shift_right_logical(v, 16)
    pltpu.async_copy(x_hbm.at[s], buf0, sem0).wait()
    pltpu.sync_copy(buf0, shared_agg.at[d], add=True)
    if with_deg:
      plsc.addupdate_scatter(deg_v, [d], one16)
    return carry

  lax.fori_loop(0, NCHUNK, step, 0)

  if with_deg:
    pltpu.sync_copy(deg_v, shared_deg.at[sid])

  plsc.subcore_barrier()

  def flush_blk(k, carry):
    rows = pl.ds(row0 + k * CHUNK, CHUNK)
    pltpu.sync_copy(shared_agg.at[rows], buf0)
    pltpu.sync_copy(buf0, agg_out.at[cid, rows])
    return carry

  lax.fori_loop(0, NBLK, flush_blk, 0)

  if with_deg:
    pltpu.sync_copy(shared_deg.at[:, pl.ds(row0, SEG)], dbuf)

    def sum_grp(q, carry):
      cols = pl.ds(q * CHUNK, CHUNK)
      acc = dbuf[0, cols]
      for r in range(1, NS):
        acc = acc + dbuf[r, cols]
      dbuf[0, cols] = acc
      return carry

    lax.fori_loop(0, NBLK, sum_grp, 0)
    pltpu.sync_copy(dbuf.at[0], deg_out.at[cid, sid])


def _make_sc_kernel(with_deg):
  mesh = plsc.VectorSubcoreMesh(core_axis_name="c", subcore_axis_name="s")
  out_type = [jax.ShapeDtypeStruct((NC, N_PAD, D), jnp.float32)]
  if with_deg:
    out_type.append(jax.ShapeDtypeStruct((NC, NS, SEG), jnp.float32))
    scratch = [
        pltpu.VMEM((E_W_PAD,), jnp.int32),
        pltpu.VMEM((CHUNK, D), jnp.float32),
        pltpu.VMEM((N_PAD,), jnp.float32),
        pltpu.VMEM((NS, SEG), jnp.float32),
        pltpu.VMEM_SHARED((N_PAD, D), jnp.float32),
        pltpu.VMEM_SHARED((NS, N_PAD), jnp.float32),
        pltpu.SemaphoreType.DMA,
    ]

    def body(x, packed, agg_out, deg_out, *s):
      _sc_agg_body(True, x, packed, agg_out, deg_out, *s)
  else:
    scratch = [
        pltpu.VMEM((E_W_PAD,), jnp.int32),
        pltpu.VMEM((CHUNK, D), jnp.float32),
        pltpu.VMEM_SHARED((N_PAD, D), jnp.float32),
        pltpu.SemaphoreType.DMA,
    ]

    def body(x, packed, agg_out, packed_v, buf0, shared_agg, sem0):
      _sc_agg_body(False, x, packed, agg_out, None,
                   packed_v, buf0, None, None, shared_agg, None, sem0)

  return pl.kernel(
      body, out_type=out_type, mesh=mesh, scratch_types=scratch,
      compiler_params=pltpu.CompilerParams(needs_layout_passes=False))


_sc_agg_deg = _make_sc_kernel(True)
_sc_agg = _make_sc_kernel(False)


def _dotT(x, w):
  return lax.dot_general(x, w, (((1,), (1,)), ((), ())),
                         preferred_element_type=jnp.float32)


R_BLK = SEG


def _inv_deg(deg_ref):
  deg = deg_ref[0] + deg_ref[1]
  inv = 1.0 / jnp.maximum(deg, 1.0)
  return inv[:, None]


def _tc_layer1_body(h_ref, agg_ref, deg_ref, ws1_ref, wn1_ref, b1_ref,
                    ws2_ref, wn2_ref, b2_ref, s2_ref, y2_ref):
  agg = agg_ref[0] + agg_ref[1]
  mean = agg * _inv_deg(deg_ref)
  x1 = _dotT(h_ref[...], ws1_ref[...]) + _dotT(mean, wn1_ref[...]) + b1_ref[...]
  x1 = jnp.maximum(x1, 0.0)
  s2_ref[...] = _dotT(x1, ws2_ref[...]) + b2_ref[...]
  y2_ref[...] = _dotT(x1, wn2_ref[...])


def _tc_layer1(h_pad, agg_p, deg_p, ws1, wn1, b1, ws2, wn2, b2):
  grid = (N_PAD // R_BLK,)

  def full(shape):
    return pl.BlockSpec(shape, lambda i: (0,) * len(shape))

  return pl.pallas_call(
      _tc_layer1_body,
      grid=grid,
      in_specs=[
          pl.BlockSpec((R_BLK, D), lambda i: (i, 0)),
          pl.BlockSpec((NC, R_BLK, D), lambda i: (0, i, 0)),
          pl.BlockSpec((NC, SEG), lambda i: (0, i)),
          full((2 * D, D)),
          full((2 * D, D)),
          full((1, 2 * D)),
          full((D, 2 * D)),
          full((D, 2 * D)),
          full((1, D)),
      ],
      out_specs=[
          pl.BlockSpec((R_BLK, D), lambda i: (i, 0)),
          pl.BlockSpec((R_BLK, D), lambda i: (i, 0)),
      ],
      out_shape=[
          jax.ShapeDtypeStruct((N_PAD, D), jnp.float32),
          jax.ShapeDtypeStruct((N_PAD, D), jnp.float32),
      ],
  )(h_pad, agg_p, deg_p, ws1, wn1, b1.reshape(1, -1), ws2, wn2,
    b2.reshape(1, -1))


def _tc_final_body(s2_ref, agg_ref, deg_ref, out_ref):
  agg = agg_ref[0] + agg_ref[1]
  mean = agg * _inv_deg(deg_ref)
  out_ref[...] = jnp.maximum(s2_ref[...] + mean, 0.0)


def _tc_final(s2, agg_p, deg_p):
  grid = (N_PAD // R_BLK,)
  return pl.pallas_call(
      _tc_final_body,
      grid=grid,
      in_specs=[
          pl.BlockSpec((R_BLK, D), lambda i: (i, 0)),
          pl.BlockSpec((NC, R_BLK, D), lambda i: (0, i, 0)),
          pl.BlockSpec((NC, SEG), lambda i: (0, i)),
      ],
      out_specs=pl.BlockSpec((R_BLK, D), lambda i: (i, 0)),
      out_shape=jax.ShapeDtypeStruct((N_PAD, D), jnp.float32),
  )(s2, agg_p, deg_p)


def kernel(h, edge_index, W_self1, W_neigh1, b1, W_self2, W_neigh2, b2):
  packed = (edge_index[0] | (edge_index[1] << 16)).reshape(NW, E_PER_W)
  pad_word = jnp.int32((N_PAD - 1) << 16)
  pad = jnp.full((NW, E_W_PAD - E_PER_W), pad_word, jnp.int32)
  packed = jnp.concatenate([packed, pad], axis=1)
  h_pad = jnp.concatenate(
      [h, jnp.zeros((N_PAD - N_NODES, D), jnp.float32)], axis=0)

  agg1_p, deg_p = _sc_agg_deg(h_pad, packed)
  deg_p = deg_p.reshape(NC, N_PAD)
  s2, y2 = _tc_layer1(h_pad, agg1_p, deg_p, W_self1, W_neigh1, b1,
                      W_self2, W_neigh2, b2)
  agg2_p, = _sc_agg(y2, packed)
  out = _tc_final(s2, agg2_p, deg_p)
  return out[:N_NODES]


# speedup vs baseline: 4.0691x; 1.6083x over previous
import jax
import jax.numpy as jnp
from jax import lax
from jax.experimental import pallas as pl
from jax.experimental.pallas import tpu as pltpu
from jax.experimental.pallas import tpu_sc as plsc

N_NODES = 10000
N_EDGES = 320000
D = 128

NC = 2
NS = 16
NW = NC * NS
E_PER_W = N_EDGES // NW
E_W_PAD = 10240
CHUNK = 16
BIG = 32
NBIG = E_W_PAD // BIG
NPAIRB = NBIG // 2 - 1
N_PAD = 10240
SEG = N_PAD // NS
NBLK = SEG // CHUNK
DQ = 4


def _sc_agg_body(with_deg, x_hbm, packed_hbm, agg_out, deg_out,
                 packed_v, buf0, buf1, srcb0, dstb0, srcb1, dstb1,
                 deg_v, dbuf, shared_agg, shared_deg, sem0, sem1):
  cid = lax.axis_index("c")
  sid = lax.axis_index("s")
  wid = cid * NS + sid
  row0 = sid * SEG

  zero16 = jnp.zeros((CHUNK,), jnp.float32)

  def zero_rows(r, carry):
    for c in range(D // CHUNK):
      buf0[r, pl.ds(c * CHUNK, CHUNK)] = zero16
    return carry

  lax.fori_loop(0, CHUNK, zero_rows, 0)

  def init_blk(k, carry):
    pltpu.sync_copy(buf0.at[pl.ds(0, CHUNK)],
                    shared_agg.at[pl.ds(row0 + k * CHUNK, CHUNK)])
    return carry

  lax.fori_loop(0, NBLK, init_blk, 0)

  if with_deg:
    def zero_deg(i, carry):
      deg_v[pl.ds(i * CHUNK, CHUNK)] = zero16
      return carry

    lax.fori_loop(0, N_PAD // CHUNK, zero_deg, 0)

  pltpu.sync_copy(packed_hbm.at[wid], packed_v)

  plsc.subcore_barrier()

  one16 = jnp.full((CHUNK,), 1.0, jnp.float32)

  def unpack(j, srcb, dstb):
    for u in range(BIG // CHUNK):
      v = packed_v[pl.ds(j * BIG + u * CHUNK, CHUNK)]
      d = lax.shift_right_logical(v, 16)
      srcb[pl.ds(u * CHUNK, CHUNK)] = v & 0xFFFF
      dstb[pl.ds(u * CHUNK, CHUNK)] = d
      if with_deg:
        plsc.addupdate_scatter(deg_v, [d], one16)

  def gather(srcb, buf, sem):
    pltpu.async_copy(x_hbm.at[srcb], buf, sem)

  def wait(srcb, buf, sem):
    pltpu.make_async_copy(x_hbm.at[srcb], buf, sem).wait()

  def scatter(dstb, buf):
    pltpu.sync_copy(buf, shared_agg.at[dstb], add=True)

  unpack(0, srcb0, dstb0)
  gather(srcb0, buf0, sem0)

  def pair(p, carry):
    j0 = 2 * p
    unpack(j0 + 1, srcb1, dstb1)
    gather(srcb1, buf1, sem1)
    wait(srcb0, buf0, sem0)
    scatter(dstb0, buf0)
    unpack(j0 + 2, srcb0, dstb0)
    gather(srcb0, buf0, sem0)
    wait(srcb1, buf1, sem1)
    scatter(dstb1, buf1)
    return carry

  lax.fori_loop(0, NPAIRB, pair, 0)
  unpack(NBIG - 1, srcb1, dstb1)
  gather(srcb1, buf1, sem1)
  wait(srcb0, buf0, sem0)
  scatter(dstb0, buf0)
  wait(srcb1, buf1, sem1)
  scatter(dstb1, buf1)

  if with_deg:
    pltpu.sync_copy(deg_v, shared_deg.at[sid])

  plsc.subcore_barrier()

  def flush_blk(k, carry):
    rows = pl.ds(row0 + k * CHUNK, CHUNK)
    pltpu.sync_copy(shared_agg.at[rows], buf0.at[pl.ds(0, CHUNK)])
    pltpu.sync_copy(buf0.at[pl.ds(0, CHUNK)], agg_out.at[cid, rows])
    return carry

  lax.fori_loop(0, NBLK, flush_blk, 0)

  if with_deg:
    for g in range(NS // DQ):
      pltpu.sync_copy(shared_deg.at[pl.ds(g * DQ, DQ), pl.ds(row0, SEG)],
                      dbuf.at[pl.ds(0, DQ)])

      def sum_q(q, carry, first=(g == 0)):
        cols = pl.ds(q * CHUNK, CHUNK)
        acc = dbuf[0, cols]
        for r in range(1, DQ):
          acc = acc + dbuf[r, cols]
        if not first:
          acc = acc + dbuf[DQ, cols]
        dbuf[DQ, cols] = acc
        return carry

      lax.fori_loop(0, NBLK, sum_q, 0)
    pltpu.sync_copy(dbuf.at[pl.ds(DQ, 1)], deg_out.at[cid, pl.ds(sid, 1)])


def _make_sc_kernel(with_deg):
  mesh = plsc.VectorSubcoreMesh(core_axis_name="c", subcore_axis_name="s")
  out_type = [jax.ShapeDtypeStruct((NC, N_PAD, D), jnp.float32)]
  common = [
      pltpu.VMEM((E_W_PAD,), jnp.int32),
      pltpu.VMEM((BIG, D), jnp.float32),
      pltpu.VMEM((BIG, D), jnp.float32),
      pltpu.VMEM((BIG,), jnp.int32),
      pltpu.VMEM((BIG,), jnp.int32),
      pltpu.VMEM((BIG,), jnp.int32),
      pltpu.VMEM((BIG,), jnp.int32),
  ]
  if with_deg:
    out_type.append(jax.ShapeDtypeStruct((NC, NS, SEG), jnp.float32))
    scratch = common + [
        pltpu.VMEM((N_PAD,), jnp.float32),
        pltpu.VMEM((DQ + 1, SEG), jnp.float32),
        pltpu.VMEM_SHARED((N_PAD, D), jnp.float32),
        pltpu.VMEM_SHARED((NS, N_PAD), jnp.float32),
        pltpu.SemaphoreType.DMA,
        pltpu.SemaphoreType.DMA,
    ]

    def body(x, packed, agg_out, deg_out, *s):
      _sc_agg_body(True, x, packed, agg_out, deg_out, *s)
  else:
    scratch = common + [
        pltpu.VMEM_SHARED((N_PAD, D), jnp.float32),
        pltpu.SemaphoreType.DMA,
        pltpu.SemaphoreType.DMA,
    ]

    def body(x, packed, agg_out, packed_v, b0, b1, s0, d0, s1, d1,
             shared_agg, sem0, sem1):
      _sc_agg_body(False, x, packed, agg_out, None,
                   packed_v, b0, b1, s0, d0, s1, d1,
                   None, None, shared_agg, None, sem0, sem1)

  return pl.kernel(
      body, out_type=out_type, mesh=mesh, scratch_types=scratch,
      compiler_params=pltpu.CompilerParams(needs_layout_passes=False))


_sc_agg_deg = _make_sc_kernel(True)
_sc_agg = _make_sc_kernel(False)


def _dotT(x, w):
  return lax.dot_general(x, w, (((1,), (1,)), ((), ())),
                         preferred_element_type=jnp.float32)


R_BLK = SEG


def _inv_deg(deg_ref):
  deg = deg_ref[0] + deg_ref[1]
  inv = 1.0 / jnp.maximum(deg, 1.0)
  return inv[:, None]


def _tc_layer1_body(h_ref, agg_ref, deg_ref, ws1_ref, wn1_ref, b1_ref,
                    ws2_ref, wn2_ref, b2_ref, s2_ref, y2_ref):
  agg = agg_ref[0] + agg_ref[1]
  mean = agg * _inv_deg(deg_ref)
  x1 = _dotT(h_ref[...], ws1_ref[...]) + _dotT(mean, wn1_ref[...]) + b1_ref[...]
  x1 = jnp.maximum(x1, 0.0)
  s2_ref[...] = _dotT(x1, ws2_ref[...]) + b2_ref[...]
  y2_ref[...] = _dotT(x1, wn2_ref[...])


def _tc_layer1(h_pad, agg_p, deg_p, ws1, wn1, b1, ws2, wn2, b2):
  grid = (N_PAD // R_BLK,)

  def full(shape):
    return pl.BlockSpec(shape, lambda i: (0,) * len(shape))

  return pl.pallas_call(
      _tc_layer1_body,
      grid=grid,
      in_specs=[
          pl.BlockSpec((R_BLK, D), lambda i: (i, 0)),
          pl.BlockSpec((NC, R_BLK, D), lambda i: (0, i, 0)),
          pl.BlockSpec((NC, SEG), lambda i: (0, i)),
          full((2 * D, D)),
          full((2 * D, D)),
          full((1, 2 * D)),
          full((D, 2 * D)),
          full((D, 2 * D)),
          full((1, D)),
      ],
      out_specs=[
          pl.BlockSpec((R_BLK, D), lambda i: (i, 0)),
          pl.BlockSpec((R_BLK, D), lambda i: (i, 0)),
      ],
      out_shape=[
          jax.ShapeDtypeStruct((N_PAD, D), jnp.float32),
          jax.ShapeDtypeStruct((N_PAD, D), jnp.float32),
      ],
  )(h_pad, agg_p, deg_p, ws1, wn1, b1.reshape(1, -1), ws2, wn2,
    b2.reshape(1, -1))


def _tc_final_body(s2_ref, agg_ref, deg_ref, out_ref):
  agg = agg_ref[0] + agg_ref[1]
  mean = agg * _inv_deg(deg_ref)
  out_ref[...] = jnp.maximum(s2_ref[...] + mean, 0.0)


def _tc_final(s2, agg_p, deg_p):
  grid = (N_PAD // R_BLK,)
  return pl.pallas_call(
      _tc_final_body,
      grid=grid,
      in_specs=[
          pl.BlockSpec((R_BLK, D), lambda i: (i, 0)),
          pl.BlockSpec((NC, R_BLK, D), lambda i: (0, i, 0)),
          pl.BlockSpec((NC, SEG), lambda i: (0, i)),
      ],
      out_specs=pl.BlockSpec((R_BLK, D), lambda i: (i, 0)),
      out_shape=jax.ShapeDtypeStruct((N_PAD, D), jnp.float32),
  )(s2, agg_p, deg_p)


def kernel(h, edge_index, W_self1, W_neigh1, b1, W_self2, W_neigh2, b2):
  packed = (edge_index[0] | (edge_index[1] << 16)).reshape(NW, E_PER_W)
  pad_word = jnp.int32((N_PAD - 1) << 16)
  pad = jnp.full((NW, E_W_PAD - E_PER_W), pad_word, jnp.int32)
  packed = jnp.concatenate([packed, pad], axis=1)
  h_pad = jnp.concatenate(
      [h, jnp.zeros((N_PAD - N_NODES, D), jnp.float32)], axis=0)

  agg1_p, deg_p = _sc_agg_deg(h_pad, packed)
  deg_p = deg_p.reshape(NC, N_PAD)
  s2, y2 = _tc_layer1(h_pad, agg1_p, deg_p, W_self1, W_neigh1, b1,
                      W_self2, W_neigh2, b2)
  agg2_p, = _sc_agg(y2, packed)
  out = _tc_final(s2, agg2_p, deg_p)
  return out[:N_NODES]
